# baseline (speedup 1.0000x reference)
import functools

import jax
import jax.numpy as jnp
from jax import lax
from jax.experimental import pallas as pl
from jax.experimental.pallas import tpu as pltpu
from jax.experimental.pallas import tpu_sc as plsc

V, D, B = 8192, 1024, 4096
_info = plsc.get_sparse_core_info()
NC, NS = _info.num_cores, _info.num_subcores
NW = NC * NS
B_PER_W = B // NW
CH = 8
NCHUNK = B_PER_W // CH
NB = 15


def _gather_kernel(table_hbm, idx_hbm, out_hbm, idx_v, *rest):
    vbufs = rest[:NB]
    gsems = rest[NB:2 * NB]
    osems = rest[2 * NB:3 * NB]
    cid = lax.axis_index("c")
    sid = lax.axis_index("s")
    wid = sid * NC + cid
    base = wid * B_PER_W
    pltpu.sync_copy(idx_hbm.at[pl.ds(base, B_PER_W)], idx_v)

    G = [None] * NCHUNK
    O = [None] * NCHUNK
    for i in range(NB):
        G[i] = pltpu.async_copy(
            table_hbm.at[idx_v.at[pl.ds(i * CH, CH)]], vbufs[i], gsems[i])
    LAG = 4
    for i in range(NCHUNK):
        b = i % NB
        G[i].wait()
        O[i] = pltpu.async_copy(
            vbufs[b], out_hbm.at[pl.ds(base + i * CH, CH)], osems[b])
        j = i - LAG
        if j >= 0 and j + NB < NCHUNK:
            O[j].wait()
            G[j + NB] = pltpu.async_copy(
                table_hbm.at[idx_v.at[pl.ds((j + NB) * CH, CH)]],
                vbufs[j % NB], gsems[j % NB])
    for i in range(NCHUNK):
        if i + NB >= NCHUNK or i > NCHUNK - 1 - LAG:
            O[i].wait()


@jax.jit
def _gather(table, idx):
    k = functools.partial(
        pl.kernel,
        mesh=plsc.VectorSubcoreMesh(core_axis_name="c", subcore_axis_name="s"),
        out_type=jax.ShapeDtypeStruct((B, D), jnp.float32),
        scratch_types=[pltpu.VMEM((B_PER_W,), jnp.int32)]
        + [pltpu.VMEM((CH, D), jnp.float32)] * NB
        + [pltpu.SemaphoreType.DMA] * (2 * NB),
    )(_gather_kernel)
    return k(table, idx)


def kernel(hidden_state, word_indices):
    table = hidden_state.reshape(V, D)
    idx = word_indices.astype(jnp.int32)
    out = _gather(table, idx)
    return out.reshape(1, B, D)


# speedup vs baseline: 1.0046x; 1.0046x over previous
import functools

import jax
import jax.numpy as jnp
from jax import lax
from jax.experimental import pallas as pl
from jax.experimental.pallas import tpu as pltpu
from jax.experimental.pallas import tpu_sc as plsc

V, D, B = 8192, 1024, 4096
_info = plsc.get_sparse_core_info()
NC, NS = _info.num_cores, _info.num_subcores
NW = NC * NS
B_PER_W = B // NW
CH = 8
NCHUNK = B_PER_W // CH
NB = 15


def _gather_kernel(table_hbm, idx_hbm, out_hbm, idx_v, *rest):
    vbufs = rest[:NB]
    gsems = rest[NB:2 * NB]
    osems = rest[2 * NB:3 * NB]
    cid = lax.axis_index("c")
    sid = lax.axis_index("s")
    wid = sid * NC + cid
    base = wid * B_PER_W
    pltpu.sync_copy(idx_hbm.at[pl.ds(base, B_PER_W)], idx_v)

    G = [None] * NCHUNK
    O = [None] * NCHUNK
    for i in range(NB):
        G[i] = pltpu.async_copy(
            table_hbm.at[idx_v.at[pl.ds(i * CH, CH)]], vbufs[i], gsems[i])
    LAG = 2
    for i in range(NCHUNK):
        b = i % NB
        G[i].wait()
        O[i] = pltpu.async_copy(
            vbufs[b], out_hbm.at[pl.ds(base + i * CH, CH)], osems[b])
        j = i - LAG
        if j >= 0 and j + NB < NCHUNK:
            O[j].wait()
            G[j + NB] = pltpu.async_copy(
                table_hbm.at[idx_v.at[pl.ds((j + NB) * CH, CH)]],
                vbufs[j % NB], gsems[j % NB])
    for i in range(NCHUNK):
        if i + NB >= NCHUNK or i > NCHUNK - 1 - LAG:
            O[i].wait()


@jax.jit
def _gather(table, idx):
    k = functools.partial(
        pl.kernel,
        mesh=plsc.VectorSubcoreMesh(core_axis_name="c", subcore_axis_name="s"),
        out_type=jax.ShapeDtypeStruct((B, D), jnp.float32),
        scratch_types=[pltpu.VMEM((B_PER_W,), jnp.int32)]
        + [pltpu.VMEM((CH, D), jnp.float32)] * NB
        + [pltpu.SemaphoreType.DMA] * (2 * NB),
    )(_gather_kernel)
    return k(table, idx)


def kernel(hidden_state, word_indices):
    table = hidden_state.reshape(V, D)
    idx = word_indices.astype(jnp.int32)
    out = _gather(table, idx)
    return out.reshape(1, B, D)
